# baseline (speedup 1.0000x reference)
import jax
import jax.numpy as jnp
from jax import lax
from jax.experimental import pallas as pl

_K = 512
_D = 32
_CHUNK = 1024


def _tree_sum_rows(v):
    gs = []
    for g in range(4):
        b = v[8 * g:8 * g + 8]
        b = b[:4] + b[4:]
        b = b[:2] + b[2:]
        b = b[:1] + b[1:]
        gs.append(b)
    return (gs[0] + gs[1]) + (gs[2] + gs[3])


def _tc_body(x_ref, c_ref, ct_ref, idx_ref, loss_ref, q_ref):
    cc = x_ref.shape[0]
    x = x_ref[...]
    c = c_ref[...]
    ct = ct_ref[...]
    cdims = (((1,), (0,)), ((), ()))

    def mm(a, b):
        return lax.dot_general(a, b, cdims,
                               preferred_element_type=jnp.float32)

    xt = jnp.transpose(x)
    xth = xt.astype(jnp.bfloat16)
    xtl = (xt - xth.astype(jnp.float32)).astype(jnp.bfloat16)
    chb = c.astype(jnp.bfloat16)
    clb = (c - chb.astype(jnp.float32)).astype(jnp.bfloat16)
    lhs_s = jnp.concatenate([chb, clb, chb], axis=1)
    rhs_s = jnp.concatenate([xth, xth, xtl], axis=0)
    cn_half = 0.5 * jnp.sum(c * c, axis=1, keepdims=True)
    st = cn_half - mm(lhs_s, rhs_s)
    iota_k = lax.broadcasted_iota(jnp.int32, st.shape, 0)
    big = jnp.float32(3e38)
    kbig = jnp.int32(_K)
    m1v = jnp.min(st, axis=0, keepdims=True)
    idx1 = jnp.min(jnp.where(st == m1v, iota_k, kbig), axis=0, keepdims=True)
    first = iota_k == idx1
    m2v = jnp.min(jnp.where(first, big, st), axis=0, keepdims=True)
    idx2 = jnp.min(jnp.where((st == m2v) & jnp.logical_not(first), iota_k,
                             kbig), axis=0, keepdims=True)
    idx12 = jnp.concatenate([idx1, idx2], axis=1)
    iota_k2 = lax.broadcasted_iota(jnp.int32, (_K, 2 * cc), 0)
    oh12 = (iota_k2 == idx12).astype(jnp.bfloat16)
    cth = ct.astype(jnp.bfloat16)
    cmf = ct - cth.astype(jnp.float32)
    cmm = cmf.astype(jnp.bfloat16)
    cll = (cmf - cmm.astype(jnp.float32)).astype(jnp.bfloat16)
    lhs_g = jnp.concatenate([cth, cmm, cll], axis=0)
    g3 = mm(lhs_g, oh12)
    c12 = (g3[:_D] + g3[_D:2 * _D]) + g3[2 * _D:]
    xt2 = jnp.concatenate([xt, xt], axis=1)
    r = xt2 - c12
    d12 = _tree_sum_rows(r * r)
    d1 = d12[:, :cc]
    d2 = d12[:, cc:]
    take1 = (d1 < d2) | ((d1 == d2) & (idx1 < idx2))
    win = jnp.where(take1, idx1, idx2)
    dmin = jnp.where(take1, d1, d2)
    idx_ref[...] = win[None]
    qt = jnp.where(take1, c12[:, :cc], c12[:, cc:])
    q_ref[...] = jnp.transpose(qt)
    part = jnp.sum(dmin, axis=1, keepdims=True) * jnp.float32(2.0 ** -20)

    @pl.when(pl.program_id(0) == 0)
    def _():
        loss_ref[...] = jnp.zeros_like(loss_ref[...])

    loss_ref[...] += part


def kernel(inputs, codewords):
    shape = inputs.shape
    n = shape[0] * shape[1]
    x = inputs.reshape(n, _D)
    ct = codewords.T
    grid = n // _CHUNK
    idx3, loss2d, q2d = pl.pallas_call(
        _tc_body,
        grid=(grid,),
        in_specs=[
            pl.BlockSpec((_CHUNK, _D), lambda i: (i, 0)),
            pl.BlockSpec((_K, _D), lambda i: (0, 0)),
            pl.BlockSpec((_D, _K), lambda i: (0, 0)),
        ],
        out_specs=[
            pl.BlockSpec((1, 1, _CHUNK), lambda i: (i, 0, 0)),
            pl.BlockSpec((1, 1), lambda i: (0, 0)),
            pl.BlockSpec((_CHUNK, _D), lambda i: (i, 0)),
        ],
        out_shape=[
            jax.ShapeDtypeStruct((grid, 1, _CHUNK), jnp.int32),
            jax.ShapeDtypeStruct((1, 1), jnp.float32),
            jax.ShapeDtypeStruct((n, _D), jnp.float32),
        ],
    )(x, codewords, ct)
    quantized = q2d.reshape(shape)
    indices = idx3.reshape(shape[:-1])
    loss = loss2d[0, 0]
    return quantized, indices, loss


# speedup vs baseline: 1.3332x; 1.3332x over previous
import jax
import jax.numpy as jnp
from jax import lax
from jax.experimental import pallas as pl

_K = 512
_D = 32
_CHUNK = 4096


def _tree_sum_rows(v):
    gs = []
    for g in range(4):
        b = v[8 * g:8 * g + 8]
        b = b[:4] + b[4:]
        b = b[:2] + b[2:]
        b = b[:1] + b[1:]
        gs.append(b)
    return (gs[0] + gs[1]) + (gs[2] + gs[3])


def _tc_body(x_ref, c_ref, ct_ref, idx_ref, loss_ref, q_ref):
    cc = x_ref.shape[0]
    x = x_ref[...]
    c = c_ref[...]
    ct = ct_ref[...]
    cdims = (((1,), (0,)), ((), ()))

    def mm(a, b):
        return lax.dot_general(a, b, cdims,
                               preferred_element_type=jnp.float32)

    xt = jnp.transpose(x)
    xth = xt.astype(jnp.bfloat16)
    xtl = (xt - xth.astype(jnp.float32)).astype(jnp.bfloat16)
    chb = c.astype(jnp.bfloat16)
    clb = (c - chb.astype(jnp.float32)).astype(jnp.bfloat16)
    lhs_s = jnp.concatenate([chb, clb, chb], axis=1)
    rhs_s = jnp.concatenate([xth, xth, xtl], axis=0)
    cn_half = 0.5 * jnp.sum(c * c, axis=1, keepdims=True)
    st = cn_half - mm(lhs_s, rhs_s)
    iota_k = lax.broadcasted_iota(jnp.int32, st.shape, 0)
    big = jnp.float32(3e38)
    kbig = jnp.int32(_K)
    idx1 = jnp.argmin(st, axis=0).astype(jnp.int32)[None, :]
    first = iota_k == idx1
    idx2 = jnp.argmin(jnp.where(first, big, st),
                      axis=0).astype(jnp.int32)[None, :]
    del kbig
    idx12 = jnp.concatenate([idx1, idx2], axis=1)
    iota_k2 = lax.broadcasted_iota(jnp.int32, (_K, 2 * cc), 0)
    oh12 = (iota_k2 == idx12).astype(jnp.bfloat16)
    cth = ct.astype(jnp.bfloat16)
    cmf = ct - cth.astype(jnp.float32)
    cmm = cmf.astype(jnp.bfloat16)
    cll = (cmf - cmm.astype(jnp.float32)).astype(jnp.bfloat16)
    lhs_g = jnp.concatenate([cth, cmm, cll], axis=0)
    g3 = mm(lhs_g, oh12)
    c12 = (g3[:_D] + g3[_D:2 * _D]) + g3[2 * _D:]
    xt2 = jnp.concatenate([xt, xt], axis=1)
    r = xt2 - c12
    d12 = _tree_sum_rows(r * r)
    d1 = d12[:, :cc]
    d2 = d12[:, cc:]
    take1 = (d1 < d2) | ((d1 == d2) & (idx1 < idx2))
    win = jnp.where(take1, idx1, idx2)
    dmin = jnp.where(take1, d1, d2)
    idx_ref[...] = win[None]
    qt = jnp.where(take1, c12[:, :cc], c12[:, cc:])
    q_ref[...] = jnp.transpose(qt)
    part = jnp.sum(dmin, axis=1, keepdims=True) * jnp.float32(2.0 ** -20)

    @pl.when(pl.program_id(0) == 0)
    def _():
        loss_ref[...] = jnp.zeros_like(loss_ref[...])

    loss_ref[...] += part


def kernel(inputs, codewords):
    shape = inputs.shape
    n = shape[0] * shape[1]
    x = inputs.reshape(n, _D)
    ct = codewords.T
    grid = n // _CHUNK
    idx3, loss2d, q2d = pl.pallas_call(
        _tc_body,
        grid=(grid,),
        in_specs=[
            pl.BlockSpec((_CHUNK, _D), lambda i: (i, 0)),
            pl.BlockSpec((_K, _D), lambda i: (0, 0)),
            pl.BlockSpec((_D, _K), lambda i: (0, 0)),
        ],
        out_specs=[
            pl.BlockSpec((1, 1, _CHUNK), lambda i: (i, 0, 0)),
            pl.BlockSpec((1, 1), lambda i: (0, 0)),
            pl.BlockSpec((_CHUNK, _D), lambda i: (i, 0)),
        ],
        out_shape=[
            jax.ShapeDtypeStruct((grid, 1, _CHUNK), jnp.int32),
            jax.ShapeDtypeStruct((1, 1), jnp.float32),
            jax.ShapeDtypeStruct((n, _D), jnp.float32),
        ],
    )(x, codewords, ct)
    quantized = q2d.reshape(shape)
    indices = idx3.reshape(shape[:-1])
    loss = loss2d[0, 0]
    return quantized, indices, loss
